# baseline (speedup 1.0000x reference)
import jax
import jax.numpy as jnp
from jax import lax
from jax.experimental import pallas as pl
from jax.experimental.pallas import tpu as pltpu
from jax.experimental.pallas import tpu_sc as plsc

B = 16384
F = 64
N_ROWS = 1000000
NC = 2
NS = 16
NW = NC * NS
BPW = B // NW
L = 16
GROUPS = BPW // L
FC = 32
NCHUNK = F // FC


def _mf_kernel(uid_hbm, iid_hbm, ut_hbm, it_hbm, ub_hbm, ib_hbm, gb_hbm,
               out_hbm,
               uid_v, iid_v, idxu_v, idxi_v, du_v, dv_v,
               ubias_v, ibias_v, acc_v, out_v, gb_v,
               sem_data, sem_bias):
    wid = lax.axis_index("s") * NC + lax.axis_index("c")
    base = wid * BPW

    pltpu.sync_copy(uid_hbm.at[pl.ds(base, BPW)], uid_v)
    pltpu.sync_copy(iid_hbm.at[pl.ds(base, BPW)], iid_v)

    cp_ub = pltpu.async_copy(ub_hbm.at[uid_v], ubias_v, sem_bias)
    cp_ib = pltpu.async_copy(ib_hbm.at[iid_v], ibias_v, sem_bias)
    pltpu.sync_copy(gb_hbm, gb_v)

    def zero_g(g, _):
        acc_v[pl.ds(g * L, L)] = jnp.zeros((L,), jnp.float32)
        return 0

    lax.fori_loop(0, GROUPS, zero_g, 0)

    for c in range(NCHUNK):
        def build_g(g, _):
            uidg = uid_v[pl.ds(g * L, L)]
            iidg = iid_v[pl.ds(g * L, L)]
            for f in range(FC):
                off = (c * FC + f) * N_ROWS
                idxu_v[pl.ds(f * BPW + g * L, L)] = uidg + off
                idxi_v[pl.ds(f * BPW + g * L, L)] = iidg + off
            return 0

        lax.fori_loop(0, GROUPS, build_g, 0)

        cp_u = pltpu.async_copy(ut_hbm.at[idxu_v], du_v, sem_data)
        cp_i = pltpu.async_copy(it_hbm.at[idxi_v], dv_v, sem_data)
        cp_u.wait()
        cp_i.wait()

        def acc_g(g, _):
            a = acc_v[pl.ds(g * L, L)]
            for f in range(FC):
                u = du_v[pl.ds(f * BPW + g * L, L)]
                v = dv_v[pl.ds(f * BPW + g * L, L)]
                a = a + u * v
            acc_v[pl.ds(g * L, L)] = a
            return 0

        lax.fori_loop(0, GROUPS, acc_g, 0)

    cp_ub.wait()
    cp_ib.wait()
    gb = gb_v[...]

    def final_g(g, _):
        x = (acc_v[pl.ds(g * L, L)] + ubias_v[pl.ds(g * L, L)]
             + ibias_v[pl.ds(g * L, L)] + gb)
        out_v[pl.ds(g * L, L)] = 1.0 / (1.0 + jnp.exp(-x))
        return 0

    lax.fori_loop(0, GROUPS, final_g, 0)

    pltpu.sync_copy(out_v, out_hbm.at[pl.ds(base, BPW)])


@jax.jit
def kernel(user_id, item_id, user_table, item_table, user_bias, item_bias,
           global_bias):
    mesh = plsc.VectorSubcoreMesh(core_axis_name="c", subcore_axis_name="s")
    run = pl.kernel(
        _mf_kernel,
        mesh=mesh,
        compiler_params=pltpu.CompilerParams(
            needs_layout_passes=False, use_tc_tiling_on_sc=False),
        out_type=jax.ShapeDtypeStruct((B,), jnp.float32),
        scratch_types=[
            pltpu.VMEM((BPW,), jnp.int32),
            pltpu.VMEM((BPW,), jnp.int32),
            pltpu.VMEM((FC * BPW,), jnp.int32),
            pltpu.VMEM((FC * BPW,), jnp.int32),
            pltpu.VMEM((FC * BPW,), jnp.float32),
            pltpu.VMEM((FC * BPW,), jnp.float32),
            pltpu.VMEM((BPW,), jnp.float32),
            pltpu.VMEM((BPW,), jnp.float32),
            pltpu.VMEM((BPW,), jnp.float32),
            pltpu.VMEM((BPW,), jnp.float32),
            pltpu.VMEM((L,), jnp.float32),
            pltpu.SemaphoreType.DMA,
            pltpu.SemaphoreType.DMA,
        ],
    )
    ut_flat = user_table.T.reshape(-1)
    it_flat = item_table.T.reshape(-1)
    return run(user_id.astype(jnp.int32), item_id.astype(jnp.int32),
               ut_flat, it_flat,
               user_bias.reshape(-1), item_bias.reshape(-1),
               jnp.broadcast_to(global_bias, (L,)))


# speedup vs baseline: 4.2178x; 4.2178x over previous
import jax
import jax.numpy as jnp
from jax import lax
from jax.experimental import pallas as pl
from jax.experimental.pallas import tpu as pltpu
from jax.experimental.pallas import tpu_sc as plsc

B = 16384
F = 64
NC = 2
NS = 16
NW = NC * NS
BPW = B // NW
L = 16
GROUPS = BPW // L


def _mf_kernel(uid_hbm, iid_hbm, ut_hbm, it_hbm, ub_hbm, ib_hbm, gb_hbm,
               out_hbm,
               uid_v, iid_v, urows_v, irows_v, ub_v, ib_v, out_v,
               gb_v, sem):
    wid = lax.axis_index("s") * NC + lax.axis_index("c")
    base = wid * BPW

    pltpu.sync_copy(uid_hbm.at[pl.ds(base, BPW)], uid_v)
    pltpu.sync_copy(iid_hbm.at[pl.ds(base, BPW)], iid_v)

    cp_u = pltpu.async_copy(ut_hbm.at[uid_v], urows_v, sem)
    cp_i = pltpu.async_copy(it_hbm.at[iid_v], irows_v, sem)
    cp_ub = pltpu.async_copy(ub_hbm.at[uid_v], ub_v, sem)
    cp_ib = pltpu.async_copy(ib_hbm.at[iid_v], ib_v, sem)

    pltpu.sync_copy(gb_hbm, gb_v)

    cp_u.wait()
    cp_i.wait()
    cp_ub.wait()
    cp_ib.wait()

    gb = gb_v[...]
    zeros16 = jnp.zeros((L,), jnp.int32)

    def group_body(g, _):
        row_idx = lax.iota(jnp.int32, L) + g * L
        acc = jnp.zeros((L,), jnp.float32)
        for f in range(F):
            col = jnp.full((L,), f, jnp.int32)
            u = plsc.load_gather(urows_v, [row_idx, col])
            v = plsc.load_gather(irows_v, [row_idx, col])
            acc = acc + u * v
        ub = plsc.load_gather(ub_v, [row_idx, zeros16])
        ib = plsc.load_gather(ib_v, [row_idx, zeros16])
        x = acc + ub + ib + gb
        p = 1.0 / (1.0 + jnp.exp(-x))
        out_v[pl.ds(g * L, L)] = p
        return 0

    lax.fori_loop(0, GROUPS, group_body, 0)

    pltpu.sync_copy(out_v, out_hbm.at[pl.ds(base, BPW)])


@jax.jit
def kernel(user_id, item_id, user_table, item_table, user_bias, item_bias,
           global_bias):
    mesh = plsc.VectorSubcoreMesh(core_axis_name="c", subcore_axis_name="s")
    run = pl.kernel(
        _mf_kernel,
        mesh=mesh,
        compiler_params=pltpu.CompilerParams(
            needs_layout_passes=False, use_tc_tiling_on_sc=False),
        out_type=jax.ShapeDtypeStruct((B,), jnp.float32),
        scratch_types=[
            pltpu.VMEM((BPW,), jnp.int32),
            pltpu.VMEM((BPW,), jnp.int32),
            pltpu.VMEM((BPW, F), jnp.float32),
            pltpu.VMEM((BPW, F), jnp.float32),
            pltpu.VMEM((BPW, L), jnp.float32),
            pltpu.VMEM((BPW, L), jnp.float32),
            pltpu.VMEM((BPW,), jnp.float32),
            pltpu.VMEM((L,), jnp.float32),
            pltpu.SemaphoreType.DMA,
        ],
    )
    ub16 = jnp.broadcast_to(user_bias, (user_bias.shape[0], L))
    ib16 = jnp.broadcast_to(item_bias, (item_bias.shape[0], L))
    return run(user_id.astype(jnp.int32), item_id.astype(jnp.int32),
               user_table, item_table, ub16, ib16,
               jnp.broadcast_to(global_bias, (L,)))


# speedup vs baseline: 8.9960x; 2.1329x over previous
import jax
import jax.numpy as jnp
from jax import lax
from jax.experimental import pallas as pl
from jax.experimental.pallas import tpu as pltpu
from jax.experimental.pallas import tpu_sc as plsc

B = 16384
F = 64
N_ROWS = 1000000
NC = 2
NS = 16
NW = NC * NS
BPW = B // NW
L = 16
GROUPS = BPW // L


def _mf_kernel(uid_hbm, iid_hbm, ut_hbm, it_hbm, ub_hbm, ib_hbm, gb_hbm,
               out_hbm,
               uid_v, iid_v, urows_v, irows_v, ubias_v, ibias_v, out_v,
               gb_v, sem):
    wid = lax.axis_index("s") * NC + lax.axis_index("c")
    base = wid * BPW

    pltpu.sync_copy(uid_hbm.at[pl.ds(base, BPW)], uid_v)
    pltpu.sync_copy(iid_hbm.at[pl.ds(base, BPW)], iid_v)

    cp_u = pltpu.async_copy(ut_hbm.at[uid_v], urows_v, sem)
    cp_i = pltpu.async_copy(it_hbm.at[iid_v], irows_v, sem)
    cp_ub = pltpu.async_copy(ub_hbm.at[0].at[uid_v], ubias_v, sem)
    cp_ib = pltpu.async_copy(ib_hbm.at[0].at[iid_v], ibias_v, sem)

    pltpu.sync_copy(gb_hbm, gb_v)

    cp_u.wait()
    cp_i.wait()
    cp_ub.wait()
    cp_ib.wait()

    gb = gb_v[...]

    def group_body(g, _):
        row_idx = lax.iota(jnp.int32, L) + g * L
        acc = jnp.zeros((L,), jnp.float32)
        for f in range(F):
            col = jnp.full((L,), f, jnp.int32)
            u = plsc.load_gather(urows_v, [row_idx, col])
            v = plsc.load_gather(irows_v, [row_idx, col])
            acc = acc + u * v
        x = (acc + ubias_v[pl.ds(g * L, L)] + ibias_v[pl.ds(g * L, L)] + gb)
        p = 1.0 / (1.0 + jnp.exp(-x))
        out_v[pl.ds(g * L, L)] = p
        return 0

    lax.fori_loop(0, GROUPS, group_body, 0)

    pltpu.sync_copy(out_v, out_hbm.at[pl.ds(base, BPW)])


@jax.jit
def kernel(user_id, item_id, user_table, item_table, user_bias, item_bias,
           global_bias):
    mesh = plsc.VectorSubcoreMesh(core_axis_name="c", subcore_axis_name="s")
    run = pl.kernel(
        _mf_kernel,
        mesh=mesh,
        compiler_params=pltpu.CompilerParams(
            needs_layout_passes=False, use_tc_tiling_on_sc=False),
        out_type=jax.ShapeDtypeStruct((B,), jnp.float32),
        scratch_types=[
            pltpu.VMEM((BPW,), jnp.int32),
            pltpu.VMEM((BPW,), jnp.int32),
            pltpu.VMEM((BPW, F), jnp.float32),
            pltpu.VMEM((BPW, F), jnp.float32),
            pltpu.VMEM((BPW,), jnp.float32),
            pltpu.VMEM((BPW,), jnp.float32),
            pltpu.VMEM((BPW,), jnp.float32),
            pltpu.VMEM((L,), jnp.float32),
            pltpu.SemaphoreType.DMA,
        ],
    )
    return run(user_id.astype(jnp.int32), item_id.astype(jnp.int32),
               user_table, item_table,
               user_bias.T, item_bias.T,
               jnp.broadcast_to(global_bias, (L,)))
